# Initial kernel scaffold; baseline (speedup 1.0000x reference)
import jax
import jax.numpy as jnp
from jax.experimental import pallas as pl


def kernel(user_latent_emb, item_latent_emb, adj_values, W1, b1, W2, b2, adj_indices, social_index, users, pos_items, neg_items):
    raise NotImplementedError("write your pallas kernel here")


# speedup vs baseline: 1.0437x; 1.0437x over previous
import jax
import jax.numpy as jnp
from jax import lax
from jax.experimental import pallas as pl
from jax.experimental.pallas import tpu as pltpu
from jax.experimental.pallas import tpu_sc as plsc

NUM_USERS = 25000
N_NODES = 50000
N_EDGES = 800000
N_SOCIAL = 200000
D = 64
H = 32
B = 1024
SIGMA = 0.25
BETA = 0.01
EDGE_BIAS = 0.5

EP = 819200
ROWS_E = 6400
RPT = 400
G = 2
NCH = RPT // G

SP = 204800
S_CHUNKS = 200

_mesh = plsc.VectorSubcoreMesh(core_axis_name="c", subcore_axis_name="s",
                               num_cores=2, num_subcores=16)

_i16 = lambda: jnp.arange(16, dtype=jnp.int32)


def _spmm_body(rows2, colsb, vals1, x, y, acc, cidx, ridx, vv, gbuf,
               semI, semG, semS):
    c = lax.axis_index("c")
    s = lax.axis_index("s")
    i16 = _i16()
    zero16 = jnp.zeros((16,), jnp.float32)

    def zb(i, carry):
        k16 = i16 + i * 16
        plsc.store_scatter(gbuf.at[0], [k16 >> 5, k16 & 31], zero16)
        return carry
    lax.fori_loop(0, (G * 128 * H) // 16, zb, 0)
    for t in range(13):
        q = s + 16 * t

        @pl.when(q < 200)
        def _():
            pltpu.sync_copy(gbuf.at[0, pl.ds(0, 250)],
                            acc.at[pl.ds(q * 250, 250)])
    plsc.subcore_barrier()

    base_row = s * RPT

    def fire_idx(j, b):
        r0 = base_row + j * G
        pltpu.async_copy(colsb.at[c, pl.ds(r0, G)], cidx.at[b], semI)
        pltpu.async_copy(rows2.at[pl.ds(r0, G)], ridx.at[b], semI)
        pltpu.async_copy(vals1.at[pl.ds(r0 * 128, G * 128)], vv.at[b], semI)

    def drain_idx(b):
        pltpu.make_async_copy(colsb.at[0, pl.ds(0, G)], cidx.at[b], semI).wait()
        pltpu.make_async_copy(rows2.at[pl.ds(0, G)], ridx.at[b], semI).wait()
        pltpu.make_async_copy(vals1.at[pl.ds(0, G * 128)], vv.at[b], semI).wait()

    def fire_gathers(b):
        for r in range(G):
            pltpu.async_copy(x.at[cidx.at[b, r]],
                             gbuf.at[b, pl.ds(r * 128, 128)], semG)

    def drain_gathers(b):
        for r in range(G):
            pltpu.make_async_copy(x.at[pl.ds(0, 128)],
                                  gbuf.at[b, pl.ds(r * 128, 128)], semG).wait()

    def scale(b):
        gb = gbuf.at[b]

        def grp(e, carry):
            vals16 = vv[b, pl.ds(e * 16, 16)]
            eid16 = i16 + e * 16
            for f in range(32):
                col16 = jnp.full((16,), f, jnp.int32)
                gv = plsc.load_gather(gb, [eid16, col16])
                plsc.store_scatter(gb, [eid16, col16], gv * vals16)
            return carry
        lax.fori_loop(0, (G * 128) // 16, grp, 0)

    def fire_scatters(b):
        for r in range(G):
            pltpu.async_copy(gbuf.at[b, pl.ds(r * 128, 128)],
                             acc.at[ridx.at[b, r]], semS, add=True)

    def drain_scatters(b):
        for r in range(G):
            pltpu.make_async_copy(x.at[pl.ds(0, 128)],
                                  gbuf.at[b, pl.ds(r * 128, 128)], semS).wait()

    fire_idx(0, 0)
    drain_idx(0)
    fire_gathers(0)

    def outer(o, carry):
        for b in range(2):
            j = o * 2 + b
            nb = 1 - b

            @pl.when(j + 1 < NCH)
            def _():
                fire_idx(j + 1, nb)
            drain_gathers(b)
            scale(b)

            @pl.when(j + 1 < NCH)
            def _():
                drain_idx(nb)
                fire_gathers(nb)
            fire_scatters(b)
            drain_scatters(b)
        return carry
    lax.fori_loop(0, NCH // 2, outer, 0)

    plsc.subcore_barrier()
    for t in range(7):
        q = s + 16 * t

        @pl.when(q < 100)
        def _():
            pltpu.sync_copy(acc.at[pl.ds(q * 500, 500)],
                            y.at[pl.ds(c * N_NODES + q * 500, 500)])


_spmm = pl.kernel(
    _spmm_body,
    out_type=jax.ShapeDtypeStruct((2 * N_NODES, H), jnp.float32),
    mesh=_mesh,
    compiler_params=pltpu.CompilerParams(needs_layout_passes=False, use_tc_tiling_on_sc=False),
    scratch_types=[
        pltpu.VMEM_SHARED((N_NODES, H), jnp.float32),
        pltpu.VMEM((2, G, 128), jnp.int32),
        pltpu.VMEM((2, G, 128), jnp.int32),
        pltpu.VMEM((2, G * 128), jnp.float32),
        pltpu.VMEM((2, G * 128, H), jnp.float32),
        pltpu.SemaphoreType.DMA,
        pltpu.SemaphoreType.DMA,
        pltpu.SemaphoreType.DMA,
    ],
)


def _edge_body(rowsb, colsb, sw1, g1, w2b, P, Q, mout,
               pbuf, qbuf, ridx, cidx, swv, gv, mbuf, w2v, sem):
    c = lax.axis_index("c")
    s = lax.axis_index("s")
    w = s * 2 + c
    i16 = _i16()
    pltpu.sync_copy(w2b, w2v)
    b2splat = w2v[pl.ds(D * 16, 16)]

    def chunk(j, carry):
        q = w + 32 * j

        @pl.when(q < S_CHUNKS)
        def _():
            e0 = q * 1024
            pltpu.sync_copy(rowsb.at[q], ridx)
            pltpu.sync_copy(colsb.at[q], cidx)
            pltpu.sync_copy(sw1.at[pl.ds(e0, 1024)], swv)
            pltpu.sync_copy(g1.at[pl.ds(e0, 1024)], gv)
            for sub in range(2):
                descs = []
                for r in range(4):
                    rr = sub * 4 + r
                    descs.append(pltpu.async_copy(
                        P.at[ridx.at[rr]], pbuf.at[pl.ds(r * 128, 128)], sem))
                    descs.append(pltpu.async_copy(
                        Q.at[cidx.at[rr]], qbuf.at[pl.ds(r * 128, 128)], sem))
                for dsc in descs:
                    dsc.wait()

                def grp(e, carry2):
                    eid16 = i16 + e * 16
                    acc = jnp.zeros((16,), jnp.float32)
                    for f in range(D):
                        col16 = jnp.full((16,), f, jnp.int32)
                        pv = plsc.load_gather(pbuf, [eid16, col16])
                        qv = plsc.load_gather(qbuf, [eid16, col16])
                        w2f = w2v[pl.ds(f * 16, 16)]
                        acc = acc + jnp.maximum(pv + qv, 0.0) * w2f
                    eo = e * 16 + sub * 512
                    gvv = gv[pl.ds(eo, 16)]
                    t = (acc + b2splat + gvv) * 5.0
                    gate = 1.0 / (1.0 + jnp.exp(-t)) + EDGE_BIAS
                    mbuf[pl.ds(eo, 16)] = swv[pl.ds(eo, 16)] * gate
                    return carry2
                lax.fori_loop(0, 32, grp, 0)
            pltpu.sync_copy(mbuf, mout.at[pl.ds(e0, 1024)])
        return carry
    lax.fori_loop(0, 7, chunk, 0)


_edge = pl.kernel(
    _edge_body,
    out_type=jax.ShapeDtypeStruct((SP,), jnp.float32),
    mesh=_mesh,
    compiler_params=pltpu.CompilerParams(needs_layout_passes=False, use_tc_tiling_on_sc=False),
    scratch_types=[
        pltpu.VMEM((512, D), jnp.float32),
        pltpu.VMEM((512, D), jnp.float32),
        pltpu.VMEM((8, 128), jnp.int32),
        pltpu.VMEM((8, 128), jnp.int32),
        pltpu.VMEM((1024,), jnp.float32),
        pltpu.VMEM((1024,), jnp.float32),
        pltpu.VMEM((1024,), jnp.float32),
        pltpu.VMEM((1280,), jnp.float32),
        pltpu.SemaphoreType.DMA,
    ],
)


def _mgather_body(ego, x1, x2, x3, y1, y2, y3, users, pos, neg,
                  uo, um, po, pm, nm,
                  ibuf, ga, gb, gc, gd, obuf, sem):
    c = lax.axis_index("c")
    s = lax.axis_index("s")
    w = s * 2 + c
    i16 = _i16()
    X = (ego, x1, x2, x3)
    Y = (ego, y1, y2, y3)
    jobs = [(users, 0, X, uo), (users, 0, Y, um),
            (pos, NUM_USERS, X, po), (pos, NUM_USERS, Y, pm),
            (neg, NUM_USERS, Y, nm)]
    for half in range(2):
        for idxref, off, tabs, outref in jobs:
            total_off = off + half * N_NODES
            pltpu.sync_copy(idxref.at[pl.ds(w * 32, 32)], ibuf)
            for h2 in range(2):
                tv = ibuf[pl.ds(h2 * 16, 16)]
                ibuf[pl.ds(h2 * 16, 16)] = tv + total_off
            descs = []
            for tab, gbuf_ in zip(tabs, (ga, gb, gc, gd)):
                descs.append(pltpu.async_copy(tab.at[ibuf], gbuf_, sem))
            for dsc in descs:
                dsc.wait()

            def grp(i, carry):
                k16 = i16 + i * 16
                r16 = k16 >> 5
                c16 = k16 & 31
                v = (plsc.load_gather(ga, [r16, c16])
                     + plsc.load_gather(gb, [r16, c16])
                     + plsc.load_gather(gc, [r16, c16])
                     + plsc.load_gather(gd, [r16, c16])) * 0.25
                plsc.store_scatter(obuf, [r16, c16], v)
                return carry
            lax.fori_loop(0, 64, grp, 0)
            pltpu.sync_copy(obuf, outref.at[pl.ds(half * B + w * 32, 32)])


_mgather = pl.kernel(
    _mgather_body,
    out_type=[jax.ShapeDtypeStruct((2 * B, H), jnp.float32)] * 5,
    mesh=_mesh,
    compiler_params=pltpu.CompilerParams(needs_layout_passes=False, use_tc_tiling_on_sc=False),
    scratch_types=[
        pltpu.VMEM((32,), jnp.int32),
        pltpu.VMEM((32, H), jnp.float32),
        pltpu.VMEM((32, H), jnp.float32),
        pltpu.VMEM((32, H), jnp.float32),
        pltpu.VMEM((32, H), jnp.float32),
        pltpu.VMEM((32, H), jnp.float32),
        pltpu.SemaphoreType.DMA,
    ],
)


def _proj_body(ego_ref, w1r_ref, w1c_ref, b1_ref, p_ref, q_ref):
    x = ego_ref[...]
    p_ref[...] = jnp.dot(x, w1r_ref[...], preferred_element_type=jnp.float32)
    q_ref[...] = (jnp.dot(x, w1c_ref[...], preferred_element_type=jnp.float32)
                  + b1_ref[...])


_proj = pl.pallas_call(
    _proj_body,
    grid=(50,),
    in_specs=[
        pl.BlockSpec((1000, D), lambda i: (i, 0)),
        pl.BlockSpec((D, D), lambda i: (0, 0)),
        pl.BlockSpec((D, D), lambda i: (0, 0)),
        pl.BlockSpec((1, D), lambda i: (0, 0)),
    ],
    out_specs=[
        pl.BlockSpec((1000, D), lambda i: (i, 0)),
        pl.BlockSpec((1000, D), lambda i: (i, 0)),
    ],
    out_shape=[jax.ShapeDtypeStruct((N_NODES, D), jnp.float32)] * 2,
)


def _tail_body(users_c, users_r, pos_c, pos_r, ue_o, ue_m, pe_o, pe_m, ne_m,
               loss_ref, auc_ref):
    uem = ue_m[...]
    pem = pe_m[...]
    nem = ne_m[...]
    pos_score = jnp.sum(uem * pem, axis=1, keepdims=True)
    neg_score = jnp.sum(uem * nem, axis=1, keepdims=True)
    dsc = neg_score - pos_score
    softplus = jnp.maximum(dsc, 0.0) + jnp.log(1.0 + jnp.exp(-jnp.abs(dsc)))
    ranking_loss = jnp.mean(softplus)
    regu_loss = 1e-4 * (jnp.sum(uem * uem) + jnp.sum(pem * pem)
                        + jnp.sum(nem * nem)) / B
    auc = jnp.mean(jnp.where(pos_score > neg_score, 1.0, 0.0))

    ri = lax.broadcasted_iota(jnp.int32, (B, B), 0)
    ci = lax.broadcasted_iota(jnp.int32, (B, B), 1)

    def first_occ(idx_c, idx_r):
        eq = (idx_c[...] == idx_r[...]) & (ci < ri)
        dup = jnp.max(jnp.where(eq, 1.0, 0.0), axis=1, keepdims=True)
        wv = 1.0 - dup
        return wv, jnp.sum(wv)

    def normalize(xv):
        nrm = jnp.sqrt(jnp.sum(xv * xv, axis=1, keepdims=True)) + 1e-12
        return xv / nrm

    def kmat(xv):
        d = lax.dot_general(xv, xv, (((1,), (1,)), ((), ())),
                            preferred_element_type=jnp.float32)
        return jnp.exp((d - 1.0) / SIGMA)

    m = float(B)

    def hsic(Kx, Ky, wc, k):
        wr = wc.reshape(1, B)
        w2 = wc * wr
        tr = jnp.sum(Kx * Ky * w2)
        mean_x = jnp.sum(Kx * w2) / (k * k)
        mean_y = jnp.sum(Ky * w2) / (k * k)
        av = lax.dot_general(Kx, wc, (((1,), (0,)), ((), ())),
                             preferred_element_type=jnp.float32)
        bv = lax.dot_general(Ky, wc, (((1,), (0,)), ((), ())),
                             preferred_element_type=jnp.float32)
        kxy = jnp.sum(wc * av * bv) / (k * k)
        hh = tr / (m * m) + mean_x * mean_y - 2.0 * kxy / m
        return hh * (m / (m - 1.0)) ** 2

    wu, ku = first_occ(users_c, users_r)
    wi, ki = first_occ(pos_c, pos_r)
    loss_user = hsic(kmat(normalize(ue_o[...])), kmat(normalize(uem)), wu, ku)
    loss_item = hsic(kmat(normalize(pe_o[...])), kmat(normalize(pem)), wi, ki)
    ib_loss = (loss_user + loss_item) * BETA

    loss_ref[...] = (ranking_loss + regu_loss + ib_loss).reshape(1, 1)
    auc_ref[...] = auc.reshape(1, 1)


_tail = pl.pallas_call(
    _tail_body,
    out_shape=[jax.ShapeDtypeStruct((1, 1), jnp.float32)] * 2,
)


def kernel(user_latent_emb, item_latent_emb, adj_values, W1, b1, W2, b2,
           adj_indices, social_index, users, pos_items, neg_items):
    i32 = jnp.int32
    f32 = jnp.float32
    ego = jnp.concatenate([user_latent_emb, item_latent_emb], axis=0)
    rows = adj_indices[:, 0]
    cols = adj_indices[:, 1]

    eps = jax.random.uniform(jax.random.key(42), (N_SOCIAL,),
                             minval=1e-6, maxval=1.0 - 1e-6, dtype=f32)
    g = jnp.log(eps) - jnp.log(1.0 - eps)

    P, Q = _proj(ego, W1[:D], W1[D:], b1.reshape(1, D))

    spad = SP - N_SOCIAL
    zs_i = jnp.zeros((spad,), i32)
    zs_f = jnp.zeros((spad,), f32)
    rows_sp = jnp.concatenate([rows[:N_SOCIAL], zs_i]).reshape(S_CHUNKS, 8, 128)
    cols_sp = jnp.concatenate([cols[:N_SOCIAL], zs_i]).reshape(S_CHUNKS, 8, 128)
    sw_p = jnp.concatenate([adj_values[:N_SOCIAL], zs_f])
    g_p = jnp.concatenate([g, zs_f])
    w2b = jnp.concatenate(
        [jnp.repeat(jnp.concatenate([W2[:, 0], b2]), 16),
         jnp.zeros((1280 - 16 * (D + 1),), f32)])
    masked_sp = _edge(rows_sp, cols_sp, sw_p, g_p, w2b, P, Q)
    masked_s = masked_sp[:N_SOCIAL]

    pad = EP - N_EDGES
    ze_i = jnp.zeros((pad,), i32)
    ze_f = jnp.zeros((pad,), f32)
    rows_p = jnp.concatenate([rows, ze_i]).reshape(ROWS_E, 128)
    cols_p = jnp.concatenate([cols, ze_i])
    colsb = jnp.stack([cols_p, cols_p + N_NODES]).reshape(2, ROWS_E, 128)
    vals_p = jnp.concatenate([adj_values, ze_f])
    mvals_p = jnp.concatenate([masked_s, adj_values[N_SOCIAL:], ze_f])

    ego_cat = jnp.concatenate([ego[:, :H], ego[:, H:]], axis=0)
    x1 = _spmm(rows_p, colsb, vals_p, ego_cat)
    x2 = _spmm(rows_p, colsb, vals_p, x1)
    x3 = _spmm(rows_p, colsb, vals_p, x2)
    y1 = _spmm(rows_p, colsb, mvals_p, ego_cat)
    y2 = _spmm(rows_p, colsb, mvals_p, y1)
    y3 = _spmm(rows_p, colsb, mvals_p, y2)

    uo, um, po, pm, nm = _mgather(ego_cat, x1, x2, x3, y1, y2, y3,
                                  users, pos_items, neg_items)

    def fix(a):
        return a.reshape(2, B, H).transpose(1, 0, 2).reshape(B, D)

    loss, auc = _tail(users.reshape(B, 1), users.reshape(1, B),
                      pos_items.reshape(B, 1), pos_items.reshape(1, B),
                      fix(uo), fix(um), fix(po), fix(pm), fix(nm))
    return loss[0, 0], auc[0, 0]


# speedup vs baseline: 4.7998x; 4.5989x over previous
import jax
import jax.numpy as jnp
from jax import lax
from jax.experimental import pallas as pl
from jax.experimental.pallas import tpu as pltpu
from jax.experimental.pallas import tpu_sc as plsc

NUM_USERS = 25000
N_NODES = 50000
N_EDGES = 800000
N_SOCIAL = 200000
D = 64
H = 32
B = 1024
SIGMA = 0.25
BETA = 0.01
EDGE_BIAS = 0.5

EP = 819200
ROWS_E = 6400
RPT = 400
G = 2
NCH = RPT // G

SP = 204800
S_CHUNKS = 200

_mesh = plsc.VectorSubcoreMesh(core_axis_name="c", subcore_axis_name="s",
                               num_cores=2, num_subcores=16)

_i16 = lambda: jnp.arange(16, dtype=jnp.int32)


def _spmm_body(rows2, colsb, vals1, x, y, acc, cidx, ridx, vv, gbuf, sbuf,
               semI, semG, semS):
    c = lax.axis_index("c")
    s = lax.axis_index("s")
    i16 = _i16()
    zero16 = jnp.zeros((16,), jnp.float32)

    def zb(i, carry):
        k16 = i16 + i * 16
        plsc.store_scatter(gbuf.at[0], [k16 >> 5, k16 & 31], zero16)
        return carry
    lax.fori_loop(0, (G * 128 * H) // 16, zb, 0)
    for t in range(13):
        q = s + 16 * t

        @pl.when(q < 200)
        def _():
            pltpu.sync_copy(gbuf.at[0, pl.ds(0, 250)],
                            acc.at[pl.ds(q * 250, 250)])
    plsc.subcore_barrier()

    base_row = s * RPT

    def fire_idx(j, b):
        r0 = base_row + j * G
        pltpu.async_copy(colsb.at[c, pl.ds(r0, G)], cidx.at[b], semI)
        pltpu.async_copy(rows2.at[pl.ds(r0, G)], ridx.at[b], semI)
        pltpu.async_copy(vals1.at[pl.ds(r0 * 128, G * 128)], vv.at[b], semI)

    def drain_idx(b):
        pltpu.make_async_copy(colsb.at[0, pl.ds(0, G)], cidx.at[b], semI).wait()
        pltpu.make_async_copy(rows2.at[pl.ds(0, G)], ridx.at[b], semI).wait()
        pltpu.make_async_copy(vals1.at[pl.ds(0, G * 128)], vv.at[b], semI).wait()

    def fire_gathers(b):
        for r in range(G):
            pltpu.async_copy(x.at[cidx.at[b, r]],
                             gbuf.at[b, pl.ds(r * 128, 128)], semG)

    def drain_gathers(b):
        for r in range(G):
            pltpu.make_async_copy(x.at[pl.ds(0, 128)],
                                  gbuf.at[b, pl.ds(r * 128, 128)], semG).wait()

    def scale(b):
        def grp(e, carry):
            vals16 = vv[b, pl.ds(e * 16, 16)]
            for k in range(16):
                idx = e * 16 + k
                vk = vals16[k]
                sbuf[idx, pl.ds(0, 16)] = gbuf[b, idx, pl.ds(0, 16)] * vk
                sbuf[idx, pl.ds(16, 16)] = gbuf[b, idx, pl.ds(16, 16)] * vk
            return carry
        lax.fori_loop(0, (G * 128) // 16, grp, 0)

    def fire_scatters(b):
        for r in range(G):
            pltpu.async_copy(sbuf.at[pl.ds(r * 128, 128)],
                             acc.at[ridx.at[b, r]], semS, add=True)

    def drain_scatters():
        for r in range(G):
            pltpu.make_async_copy(x.at[pl.ds(0, 128)],
                                  sbuf.at[pl.ds(r * 128, 128)], semS).wait()

    fire_idx(0, 0)
    drain_idx(0)
    fire_gathers(0)
    fire_idx(1, 1)

    def outer(o, carry):
        for b in range(2):
            j = o * 2 + b
            nb = 1 - b

            @pl.when(j + 1 < NCH)
            def _():
                drain_idx(nb)
                fire_gathers(nb)
            drain_gathers(b)
            scale(b)

            @pl.when(j + 2 < NCH)
            def _():
                fire_idx(j + 2, b)
            fire_scatters(b)
            drain_scatters()
        return carry
    lax.fori_loop(0, NCH // 2, outer, 0)

    plsc.subcore_barrier()
    for t in range(7):
        q = s + 16 * t

        @pl.when(q < 100)
        def _():
            pltpu.sync_copy(acc.at[pl.ds(q * 500, 500)],
                            y.at[pl.ds(c * N_NODES + q * 500, 500)])


_spmm = pl.kernel(
    _spmm_body,
    out_type=jax.ShapeDtypeStruct((2 * N_NODES, H), jnp.float32),
    mesh=_mesh,
    compiler_params=pltpu.CompilerParams(needs_layout_passes=False, use_tc_tiling_on_sc=False),
    scratch_types=[
        pltpu.VMEM_SHARED((N_NODES, H), jnp.float32),
        pltpu.VMEM((2, G, 128), jnp.int32),
        pltpu.VMEM((2, G, 128), jnp.int32),
        pltpu.VMEM((2, G * 128), jnp.float32),
        pltpu.VMEM((2, G * 128, H), jnp.float32),
        pltpu.VMEM((G * 128, H), jnp.float32),
        pltpu.SemaphoreType.DMA,
        pltpu.SemaphoreType.DMA,
        pltpu.SemaphoreType.DMA,
    ],
)


def _edge_body(rowsb, colsb, sw1, g1, w2b, P, Q, mout,
               pbuf, qbuf, ridx, cidx, swv, gv, mbuf, w2v, sem):
    c = lax.axis_index("c")
    s = lax.axis_index("s")
    w = s * 2 + c
    i16 = _i16()
    pltpu.sync_copy(w2b, w2v)
    b2splat = w2v[pl.ds(D * 16, 16)]

    def chunk(j, carry):
        q = w + 32 * j

        @pl.when(q < S_CHUNKS)
        def _():
            e0 = q * 1024
            pltpu.sync_copy(rowsb.at[q], ridx)
            pltpu.sync_copy(colsb.at[q], cidx)
            pltpu.sync_copy(sw1.at[pl.ds(e0, 1024)], swv)
            pltpu.sync_copy(g1.at[pl.ds(e0, 1024)], gv)
            for sub in range(2):
                descs = []
                for r in range(4):
                    rr = sub * 4 + r
                    descs.append(pltpu.async_copy(
                        P.at[ridx.at[rr]], pbuf.at[pl.ds(r * 128, 128)], sem))
                    descs.append(pltpu.async_copy(
                        Q.at[cidx.at[rr]], qbuf.at[pl.ds(r * 128, 128)], sem))
                for dsc in descs:
                    dsc.wait()

                def grp(e, carry2):
                    eid16 = i16 + e * 16
                    acc = jnp.zeros((16,), jnp.float32)
                    for f in range(D):
                        col16 = jnp.full((16,), f, jnp.int32)
                        pv = plsc.load_gather(pbuf, [eid16, col16])
                        qv = plsc.load_gather(qbuf, [eid16, col16])
                        w2f = w2v[pl.ds(f * 16, 16)]
                        acc = acc + jnp.maximum(pv + qv, 0.0) * w2f
                    eo = e * 16 + sub * 512
                    gvv = gv[pl.ds(eo, 16)]
                    t = (acc + b2splat + gvv) * 5.0
                    gate = 1.0 / (1.0 + jnp.exp(-t)) + EDGE_BIAS
                    mbuf[pl.ds(eo, 16)] = swv[pl.ds(eo, 16)] * gate
                    return carry2
                lax.fori_loop(0, 32, grp, 0)
            pltpu.sync_copy(mbuf, mout.at[pl.ds(e0, 1024)])
        return carry
    lax.fori_loop(0, 7, chunk, 0)


_edge = pl.kernel(
    _edge_body,
    out_type=jax.ShapeDtypeStruct((SP,), jnp.float32),
    mesh=_mesh,
    compiler_params=pltpu.CompilerParams(needs_layout_passes=False, use_tc_tiling_on_sc=False),
    scratch_types=[
        pltpu.VMEM((512, D), jnp.float32),
        pltpu.VMEM((512, D), jnp.float32),
        pltpu.VMEM((8, 128), jnp.int32),
        pltpu.VMEM((8, 128), jnp.int32),
        pltpu.VMEM((1024,), jnp.float32),
        pltpu.VMEM((1024,), jnp.float32),
        pltpu.VMEM((1024,), jnp.float32),
        pltpu.VMEM((1280,), jnp.float32),
        pltpu.SemaphoreType.DMA,
    ],
)


def _mgather_body(ego, x1, x2, x3, y1, y2, y3, users, pos, neg,
                  uo, um, po, pm, nm,
                  ibuf, ga, gb, gc, gd, obuf, sem):
    c = lax.axis_index("c")
    s = lax.axis_index("s")
    w = s * 2 + c
    i16 = _i16()
    X = (ego, x1, x2, x3)
    Y = (ego, y1, y2, y3)
    jobs = [(users, 0, X, uo), (users, 0, Y, um),
            (pos, NUM_USERS, X, po), (pos, NUM_USERS, Y, pm),
            (neg, NUM_USERS, Y, nm)]
    for half in range(2):
        for idxref, off, tabs, outref in jobs:
            total_off = off + half * N_NODES
            pltpu.sync_copy(idxref.at[pl.ds(w * 32, 32)], ibuf)
            for h2 in range(2):
                tv = ibuf[pl.ds(h2 * 16, 16)]
                ibuf[pl.ds(h2 * 16, 16)] = tv + total_off
            descs = []
            for tab, gbuf_ in zip(tabs, (ga, gb, gc, gd)):
                descs.append(pltpu.async_copy(tab.at[ibuf], gbuf_, sem))
            for dsc in descs:
                dsc.wait()

            def grp(i, carry):
                k16 = i16 + i * 16
                r16 = k16 >> 5
                c16 = k16 & 31
                v = (plsc.load_gather(ga, [r16, c16])
                     + plsc.load_gather(gb, [r16, c16])
                     + plsc.load_gather(gc, [r16, c16])
                     + plsc.load_gather(gd, [r16, c16])) * 0.25
                plsc.store_scatter(obuf, [r16, c16], v)
                return carry
            lax.fori_loop(0, 64, grp, 0)
            pltpu.sync_copy(obuf, outref.at[pl.ds(half * B + w * 32, 32)])


_mgather = pl.kernel(
    _mgather_body,
    out_type=[jax.ShapeDtypeStruct((2 * B, H), jnp.float32)] * 5,
    mesh=_mesh,
    compiler_params=pltpu.CompilerParams(needs_layout_passes=False, use_tc_tiling_on_sc=False),
    scratch_types=[
        pltpu.VMEM((32,), jnp.int32),
        pltpu.VMEM((32, H), jnp.float32),
        pltpu.VMEM((32, H), jnp.float32),
        pltpu.VMEM((32, H), jnp.float32),
        pltpu.VMEM((32, H), jnp.float32),
        pltpu.VMEM((32, H), jnp.float32),
        pltpu.SemaphoreType.DMA,
    ],
)


def _proj_body(ego_ref, w1r_ref, w1c_ref, b1_ref, p_ref, q_ref):
    x = ego_ref[...]
    p_ref[...] = jnp.dot(x, w1r_ref[...], preferred_element_type=jnp.float32)
    q_ref[...] = (jnp.dot(x, w1c_ref[...], preferred_element_type=jnp.float32)
                  + b1_ref[...])


_proj = pl.pallas_call(
    _proj_body,
    grid=(50,),
    in_specs=[
        pl.BlockSpec((1000, D), lambda i: (i, 0)),
        pl.BlockSpec((D, D), lambda i: (0, 0)),
        pl.BlockSpec((D, D), lambda i: (0, 0)),
        pl.BlockSpec((1, D), lambda i: (0, 0)),
    ],
    out_specs=[
        pl.BlockSpec((1000, D), lambda i: (i, 0)),
        pl.BlockSpec((1000, D), lambda i: (i, 0)),
    ],
    out_shape=[jax.ShapeDtypeStruct((N_NODES, D), jnp.float32)] * 2,
)


def _tail_body(users_c, users_r, pos_c, pos_r, ue_o, ue_m, pe_o, pe_m, ne_m,
               loss_ref, auc_ref):
    uem = ue_m[...]
    pem = pe_m[...]
    nem = ne_m[...]
    pos_score = jnp.sum(uem * pem, axis=1, keepdims=True)
    neg_score = jnp.sum(uem * nem, axis=1, keepdims=True)
    dsc = neg_score - pos_score
    softplus = jnp.maximum(dsc, 0.0) + jnp.log(1.0 + jnp.exp(-jnp.abs(dsc)))
    ranking_loss = jnp.mean(softplus)
    regu_loss = 1e-4 * (jnp.sum(uem * uem) + jnp.sum(pem * pem)
                        + jnp.sum(nem * nem)) / B
    auc = jnp.mean(jnp.where(pos_score > neg_score, 1.0, 0.0))

    ri = lax.broadcasted_iota(jnp.int32, (B, B), 0)
    ci = lax.broadcasted_iota(jnp.int32, (B, B), 1)

    def first_occ(idx_c, idx_r):
        eq = (idx_c[...] == idx_r[...]) & (ci < ri)
        dup = jnp.max(jnp.where(eq, 1.0, 0.0), axis=1, keepdims=True)
        wv = 1.0 - dup
        return wv, jnp.sum(wv)

    def normalize(xv):
        nrm = jnp.sqrt(jnp.sum(xv * xv, axis=1, keepdims=True)) + 1e-12
        return xv / nrm

    def kmat(xv):
        d = lax.dot_general(xv, xv, (((1,), (1,)), ((), ())),
                            preferred_element_type=jnp.float32)
        return jnp.exp((d - 1.0) / SIGMA)

    m = float(B)

    def hsic(Kx, Ky, wc, k):
        wr = wc.reshape(1, B)
        w2 = wc * wr
        tr = jnp.sum(Kx * Ky * w2)
        mean_x = jnp.sum(Kx * w2) / (k * k)
        mean_y = jnp.sum(Ky * w2) / (k * k)
        av = lax.dot_general(Kx, wc, (((1,), (0,)), ((), ())),
                             preferred_element_type=jnp.float32)
        bv = lax.dot_general(Ky, wc, (((1,), (0,)), ((), ())),
                             preferred_element_type=jnp.float32)
        kxy = jnp.sum(wc * av * bv) / (k * k)
        hh = tr / (m * m) + mean_x * mean_y - 2.0 * kxy / m
        return hh * (m / (m - 1.0)) ** 2

    wu, ku = first_occ(users_c, users_r)
    wi, ki = first_occ(pos_c, pos_r)
    loss_user = hsic(kmat(normalize(ue_o[...])), kmat(normalize(uem)), wu, ku)
    loss_item = hsic(kmat(normalize(pe_o[...])), kmat(normalize(pem)), wi, ki)
    ib_loss = (loss_user + loss_item) * BETA

    loss_ref[...] = (ranking_loss + regu_loss + ib_loss).reshape(1, 1)
    auc_ref[...] = auc.reshape(1, 1)


_tail = pl.pallas_call(
    _tail_body,
    out_shape=[jax.ShapeDtypeStruct((1, 1), jnp.float32)] * 2,
)


def kernel(user_latent_emb, item_latent_emb, adj_values, W1, b1, W2, b2,
           adj_indices, social_index, users, pos_items, neg_items):
    i32 = jnp.int32
    f32 = jnp.float32
    ego = jnp.concatenate([user_latent_emb, item_latent_emb], axis=0)
    rows = adj_indices[:, 0]
    cols = adj_indices[:, 1]

    eps = jax.random.uniform(jax.random.key(42), (N_SOCIAL,),
                             minval=1e-6, maxval=1.0 - 1e-6, dtype=f32)
    g = jnp.log(eps) - jnp.log(1.0 - eps)

    P, Q = _proj(ego, W1[:D], W1[D:], b1.reshape(1, D))

    spad = SP - N_SOCIAL
    zs_i = jnp.zeros((spad,), i32)
    zs_f = jnp.zeros((spad,), f32)
    rows_sp = jnp.concatenate([rows[:N_SOCIAL], zs_i]).reshape(S_CHUNKS, 8, 128)
    cols_sp = jnp.concatenate([cols[:N_SOCIAL], zs_i]).reshape(S_CHUNKS, 8, 128)
    sw_p = jnp.concatenate([adj_values[:N_SOCIAL], zs_f])
    g_p = jnp.concatenate([g, zs_f])
    w2b = jnp.concatenate(
        [jnp.repeat(jnp.concatenate([W2[:, 0], b2]), 16),
         jnp.zeros((1280 - 16 * (D + 1),), f32)])
    masked_sp = _edge(rows_sp, cols_sp, sw_p, g_p, w2b, P, Q)
    masked_s = masked_sp[:N_SOCIAL]

    pad = EP - N_EDGES
    ze_i = jnp.zeros((pad,), i32)
    ze_f = jnp.zeros((pad,), f32)
    rows_p = jnp.concatenate([rows, ze_i]).reshape(ROWS_E, 128)
    cols_p = jnp.concatenate([cols, ze_i])
    colsb = jnp.stack([cols_p, cols_p + N_NODES]).reshape(2, ROWS_E, 128)
    vals_p = jnp.concatenate([adj_values, ze_f])
    mvals_p = jnp.concatenate([masked_s, adj_values[N_SOCIAL:], ze_f])

    ego_cat = jnp.concatenate([ego[:, :H], ego[:, H:]], axis=0)
    x1 = _spmm(rows_p, colsb, vals_p, ego_cat)
    x2 = _spmm(rows_p, colsb, vals_p, x1)
    x3 = _spmm(rows_p, colsb, vals_p, x2)
    y1 = _spmm(rows_p, colsb, mvals_p, ego_cat)
    y2 = _spmm(rows_p, colsb, mvals_p, y1)
    y3 = _spmm(rows_p, colsb, mvals_p, y2)

    uo, um, po, pm, nm = _mgather(ego_cat, x1, x2, x3, y1, y2, y3,
                                  users, pos_items, neg_items)

    def fix(a):
        return a.reshape(2, B, H).transpose(1, 0, 2).reshape(B, D)

    loss, auc = _tail(users.reshape(B, 1), users.reshape(1, B),
                      pos_items.reshape(B, 1), pos_items.reshape(1, B),
                      fix(uo), fix(um), fix(po), fix(pm), fix(nm))
    return loss[0, 0], auc[0, 0]


# speedup vs baseline: 5.3670x; 1.1182x over previous
import jax
import jax.numpy as jnp
from jax import lax
from jax.experimental import pallas as pl
from jax.experimental.pallas import tpu as pltpu
from jax.experimental.pallas import tpu_sc as plsc

NUM_USERS = 25000
N_NODES = 50000
N_EDGES = 800000
N_SOCIAL = 200000
D = 64
H = 32
B = 1024
SIGMA = 0.25
BETA = 0.01
EDGE_BIAS = 0.5

EP = 819200
ROWS_E = 6400
RPT = 400
G = 2
NCH = RPT // G

SP = 204800
S_CHUNKS = 200

_mesh = plsc.VectorSubcoreMesh(core_axis_name="c", subcore_axis_name="s",
                               num_cores=2, num_subcores=16)

_i16 = lambda: jnp.arange(16, dtype=jnp.int32)


def _spmm_body(rows2, colsb, vals1, x, y, acc, cidx, ridx, vv, gbuf, sbuf,
               semI, semG, semS):
    c = lax.axis_index("c")
    s = lax.axis_index("s")
    i16 = _i16()
    zero16 = jnp.zeros((16,), jnp.float32)

    def zb(i, carry):
        k16 = i16 + i * 16
        plsc.store_scatter(gbuf.at[0], [k16 >> 5, k16 & 31], zero16)
        return carry
    lax.fori_loop(0, (G * 128 * H) // 16, zb, 0)
    for t in range(13):
        q = s + 16 * t

        @pl.when(q < 200)
        def _():
            pltpu.sync_copy(gbuf.at[0, pl.ds(0, 250)],
                            acc.at[pl.ds(q * 250, 250)])
    plsc.subcore_barrier()

    base_row = s * RPT

    def fire_idx(j, b):
        r0 = base_row + j * G
        pltpu.async_copy(colsb.at[c, pl.ds(r0, G)], cidx.at[b], semI)
        pltpu.async_copy(rows2.at[pl.ds(r0, G)], ridx.at[b], semI)
        pltpu.async_copy(vals1.at[pl.ds(r0 * 128, G * 128)], vv.at[b], semI)

    def drain_idx(b):
        pltpu.make_async_copy(colsb.at[0, pl.ds(0, G)], cidx.at[b], semI).wait()
        pltpu.make_async_copy(rows2.at[pl.ds(0, G)], ridx.at[b], semI).wait()
        pltpu.make_async_copy(vals1.at[pl.ds(0, G * 128)], vv.at[b], semI).wait()

    def fire_gathers(b):
        for r in range(G):
            pltpu.async_copy(x.at[cidx.at[b, r]],
                             gbuf.at[b, pl.ds(r * 128, 128)], semG)

    def drain_gathers(b):
        for r in range(G):
            pltpu.make_async_copy(x.at[pl.ds(0, 128)],
                                  gbuf.at[b, pl.ds(r * 128, 128)], semG).wait()

    def scale(b):
        def grp(e, carry):
            vals16 = vv[b, pl.ds(e * 16, 16)]
            for k in range(16):
                idx = e * 16 + k
                vk = vals16[k]
                sbuf[idx, pl.ds(0, 16)] = gbuf[b, idx, pl.ds(0, 16)] * vk
                sbuf[idx, pl.ds(16, 16)] = gbuf[b, idx, pl.ds(16, 16)] * vk
            return carry
        lax.fori_loop(0, (G * 128) // 16, grp, 0)

    def fire_scatters(b):
        for r in range(G):
            pltpu.async_copy(sbuf.at[pl.ds(r * 128, 128)],
                             acc.at[ridx.at[b, r]], semS, add=True)

    def drain_scatters():
        for r in range(G):
            pltpu.make_async_copy(x.at[pl.ds(0, 128)],
                                  sbuf.at[pl.ds(r * 128, 128)], semS).wait()

    fire_idx(0, 0)
    drain_idx(0)
    fire_gathers(0)
    fire_idx(1, 1)

    def outer(o, carry):
        for b in range(2):
            j = o * 2 + b
            nb = 1 - b

            @pl.when(j + 1 < NCH)
            def _():
                drain_idx(nb)
                fire_gathers(nb)
            drain_gathers(b)
            scale(b)

            @pl.when(j + 2 < NCH)
            def _():
                fire_idx(j + 2, b)
            fire_scatters(b)
            drain_scatters()
        return carry
    lax.fori_loop(0, NCH // 2, outer, 0)

    plsc.subcore_barrier()
    for t in range(7):
        q = s + 16 * t

        @pl.when(q < 100)
        def _():
            pltpu.sync_copy(acc.at[pl.ds(q * 500, 500)],
                            y.at[pl.ds(c * N_NODES + q * 500, 500)])


_spmm = pl.kernel(
    _spmm_body,
    out_type=jax.ShapeDtypeStruct((2 * N_NODES, H), jnp.float32),
    mesh=_mesh,
    compiler_params=pltpu.CompilerParams(needs_layout_passes=False, use_tc_tiling_on_sc=False),
    scratch_types=[
        pltpu.VMEM_SHARED((N_NODES, H), jnp.float32),
        pltpu.VMEM((2, G, 128), jnp.int32),
        pltpu.VMEM((2, G, 128), jnp.int32),
        pltpu.VMEM((2, G * 128), jnp.float32),
        pltpu.VMEM((2, G * 128, H), jnp.float32),
        pltpu.VMEM((G * 128, H), jnp.float32),
        pltpu.SemaphoreType.DMA,
        pltpu.SemaphoreType.DMA,
        pltpu.SemaphoreType.DMA,
    ],
)


def _edge_body(rowsb, colsb, sw1, g1, w2b, P, Q, mout,
               pbuf, qbuf, ridx, cidx, swv, gv, mbuf, w2v, sem):
    c = lax.axis_index("c")
    s = lax.axis_index("s")
    w = s * 2 + c
    i16 = _i16()
    pltpu.sync_copy(w2b, w2v)
    w2r = [w2v[pl.ds(j * 16, 16)] for j in range(4)]
    b2splat = w2v[pl.ds(D, 16)]
    lmask = [i16 == k for k in range(16)]

    def chunk(j, carry):
        q = w + 32 * j

        @pl.when(q < S_CHUNKS)
        def _():
            e0 = q * 1024
            pltpu.sync_copy(rowsb.at[q], ridx)
            pltpu.sync_copy(colsb.at[q], cidx)
            pltpu.sync_copy(sw1.at[pl.ds(e0, 1024)], swv)
            pltpu.sync_copy(g1.at[pl.ds(e0, 1024)], gv)
            for sub in range(2):
                descs = []
                for r in range(4):
                    rr = sub * 4 + r
                    descs.append(pltpu.async_copy(
                        P.at[ridx.at[rr]], pbuf.at[pl.ds(r * 128, 128)], sem))
                    descs.append(pltpu.async_copy(
                        Q.at[cidx.at[rr]], qbuf.at[pl.ds(r * 128, 128)], sem))
                for dsc in descs:
                    dsc.wait()

                def grp(e, carry2):
                    acc = jnp.zeros((16,), jnp.float32)
                    for k in range(16):
                        idx = e * 16 + k
                        sv = jnp.zeros((16,), jnp.float32)
                        for j in range(4):
                            pv = pbuf[idx, pl.ds(j * 16, 16)]
                            qv = qbuf[idx, pl.ds(j * 16, 16)]
                            sv = sv + jnp.maximum(pv + qv, 0.0) * w2r[j]
                        lg = lax.reduce_sum(sv, (0,))
                        acc = jnp.where(lmask[k], lg, acc)
                    eo = e * 16 + sub * 512
                    gvv = gv[pl.ds(eo, 16)]
                    t = (acc + b2splat + gvv) * 5.0
                    gate = 1.0 / (1.0 + jnp.exp(-t)) + EDGE_BIAS
                    mbuf[pl.ds(eo, 16)] = swv[pl.ds(eo, 16)] * gate
                    return carry2
                lax.fori_loop(0, 32, grp, 0)
            pltpu.sync_copy(mbuf, mout.at[pl.ds(e0, 1024)])
        return carry
    lax.fori_loop(0, 7, chunk, 0)


_edge = pl.kernel(
    _edge_body,
    out_type=jax.ShapeDtypeStruct((SP,), jnp.float32),
    mesh=_mesh,
    compiler_params=pltpu.CompilerParams(needs_layout_passes=False, use_tc_tiling_on_sc=False),
    scratch_types=[
        pltpu.VMEM((512, D), jnp.float32),
        pltpu.VMEM((512, D), jnp.float32),
        pltpu.VMEM((8, 128), jnp.int32),
        pltpu.VMEM((8, 128), jnp.int32),
        pltpu.VMEM((1024,), jnp.float32),
        pltpu.VMEM((1024,), jnp.float32),
        pltpu.VMEM((1024,), jnp.float32),
        pltpu.VMEM((1280,), jnp.float32),
        pltpu.SemaphoreType.DMA,
    ],
)


def _mgather_body(ego, x1, x2, x3, y1, y2, y3, users, pos, neg,
                  uo, um, po, pm, nm,
                  ibuf, ga, gb, gc, gd, obuf, sem):
    c = lax.axis_index("c")
    s = lax.axis_index("s")
    w = s * 2 + c
    i16 = _i16()
    X = (ego, x1, x2, x3)
    Y = (ego, y1, y2, y3)
    jobs = [(users, 0, X, uo), (users, 0, Y, um),
            (pos, NUM_USERS, X, po), (pos, NUM_USERS, Y, pm),
            (neg, NUM_USERS, Y, nm)]
    for half in range(2):
        for idxref, off, tabs, outref in jobs:
            total_off = off + half * N_NODES
            pltpu.sync_copy(idxref.at[pl.ds(w * 32, 32)], ibuf)
            for h2 in range(2):
                tv = ibuf[pl.ds(h2 * 16, 16)]
                ibuf[pl.ds(h2 * 16, 16)] = tv + total_off
            descs = []
            for tab, gbuf_ in zip(tabs, (ga, gb, gc, gd)):
                descs.append(pltpu.async_copy(tab.at[ibuf], gbuf_, sem))
            for dsc in descs:
                dsc.wait()

            def grp(i, carry):
                k16 = i16 + i * 16
                r16 = k16 >> 5
                c16 = k16 & 31
                v = (plsc.load_gather(ga, [r16, c16])
                     + plsc.load_gather(gb, [r16, c16])
                     + plsc.load_gather(gc, [r16, c16])
                     + plsc.load_gather(gd, [r16, c16])) * 0.25
                plsc.store_scatter(obuf, [r16, c16], v)
                return carry
            lax.fori_loop(0, 64, grp, 0)
            pltpu.sync_copy(obuf, outref.at[pl.ds(half * B + w * 32, 32)])


_mgather = pl.kernel(
    _mgather_body,
    out_type=[jax.ShapeDtypeStruct((2 * B, H), jnp.float32)] * 5,
    mesh=_mesh,
    compiler_params=pltpu.CompilerParams(needs_layout_passes=False, use_tc_tiling_on_sc=False),
    scratch_types=[
        pltpu.VMEM((32,), jnp.int32),
        pltpu.VMEM((32, H), jnp.float32),
        pltpu.VMEM((32, H), jnp.float32),
        pltpu.VMEM((32, H), jnp.float32),
        pltpu.VMEM((32, H), jnp.float32),
        pltpu.VMEM((32, H), jnp.float32),
        pltpu.SemaphoreType.DMA,
    ],
)


def _proj_body(ego_ref, w1r_ref, w1c_ref, b1_ref, p_ref, q_ref):
    x = ego_ref[...]
    p_ref[...] = jnp.dot(x, w1r_ref[...], preferred_element_type=jnp.float32)
    q_ref[...] = (jnp.dot(x, w1c_ref[...], preferred_element_type=jnp.float32)
                  + b1_ref[...])


_proj = pl.pallas_call(
    _proj_body,
    grid=(50,),
    in_specs=[
        pl.BlockSpec((1000, D), lambda i: (i, 0)),
        pl.BlockSpec((D, D), lambda i: (0, 0)),
        pl.BlockSpec((D, D), lambda i: (0, 0)),
        pl.BlockSpec((1, D), lambda i: (0, 0)),
    ],
    out_specs=[
        pl.BlockSpec((1000, D), lambda i: (i, 0)),
        pl.BlockSpec((1000, D), lambda i: (i, 0)),
    ],
    out_shape=[jax.ShapeDtypeStruct((N_NODES, D), jnp.float32)] * 2,
)


def _tail_body(users_c, users_r, pos_c, pos_r, ue_o, ue_m, pe_o, pe_m, ne_m,
               loss_ref, auc_ref):
    uem = ue_m[...]
    pem = pe_m[...]
    nem = ne_m[...]
    pos_score = jnp.sum(uem * pem, axis=1, keepdims=True)
    neg_score = jnp.sum(uem * nem, axis=1, keepdims=True)
    dsc = neg_score - pos_score
    softplus = jnp.maximum(dsc, 0.0) + jnp.log(1.0 + jnp.exp(-jnp.abs(dsc)))
    ranking_loss = jnp.mean(softplus)
    regu_loss = 1e-4 * (jnp.sum(uem * uem) + jnp.sum(pem * pem)
                        + jnp.sum(nem * nem)) / B
    auc = jnp.mean(jnp.where(pos_score > neg_score, 1.0, 0.0))

    ri = lax.broadcasted_iota(jnp.int32, (B, B), 0)
    ci = lax.broadcasted_iota(jnp.int32, (B, B), 1)

    def first_occ(idx_c, idx_r):
        eq = (idx_c[...] == idx_r[...]) & (ci < ri)
        dup = jnp.max(jnp.where(eq, 1.0, 0.0), axis=1, keepdims=True)
        wv = 1.0 - dup
        return wv, jnp.sum(wv)

    def normalize(xv):
        nrm = jnp.sqrt(jnp.sum(xv * xv, axis=1, keepdims=True)) + 1e-12
        return xv / nrm

    def kmat(xv):
        d = lax.dot_general(xv, xv, (((1,), (1,)), ((), ())),
                            preferred_element_type=jnp.float32)
        return jnp.exp((d - 1.0) / SIGMA)

    m = float(B)

    def hsic(Kx, Ky, wc, k):
        wr = wc.reshape(1, B)
        w2 = wc * wr
        tr = jnp.sum(Kx * Ky * w2)
        mean_x = jnp.sum(Kx * w2) / (k * k)
        mean_y = jnp.sum(Ky * w2) / (k * k)
        av = lax.dot_general(Kx, wc, (((1,), (0,)), ((), ())),
                             preferred_element_type=jnp.float32)
        bv = lax.dot_general(Ky, wc, (((1,), (0,)), ((), ())),
                             preferred_element_type=jnp.float32)
        kxy = jnp.sum(wc * av * bv) / (k * k)
        hh = tr / (m * m) + mean_x * mean_y - 2.0 * kxy / m
        return hh * (m / (m - 1.0)) ** 2

    wu, ku = first_occ(users_c, users_r)
    wi, ki = first_occ(pos_c, pos_r)
    loss_user = hsic(kmat(normalize(ue_o[...])), kmat(normalize(uem)), wu, ku)
    loss_item = hsic(kmat(normalize(pe_o[...])), kmat(normalize(pem)), wi, ki)
    ib_loss = (loss_user + loss_item) * BETA

    loss_ref[...] = (ranking_loss + regu_loss + ib_loss).reshape(1, 1)
    auc_ref[...] = auc.reshape(1, 1)


_tail = pl.pallas_call(
    _tail_body,
    out_shape=[jax.ShapeDtypeStruct((1, 1), jnp.float32)] * 2,
)


def kernel(user_latent_emb, item_latent_emb, adj_values, W1, b1, W2, b2,
           adj_indices, social_index, users, pos_items, neg_items):
    i32 = jnp.int32
    f32 = jnp.float32
    ego = jnp.concatenate([user_latent_emb, item_latent_emb], axis=0)
    rows = adj_indices[:, 0]
    cols = adj_indices[:, 1]

    eps = jax.random.uniform(jax.random.key(42), (N_SOCIAL,),
                             minval=1e-6, maxval=1.0 - 1e-6, dtype=f32)
    g = jnp.log(eps) - jnp.log(1.0 - eps)

    P, Q = _proj(ego, W1[:D], W1[D:], b1.reshape(1, D))

    spad = SP - N_SOCIAL
    zs_i = jnp.zeros((spad,), i32)
    zs_f = jnp.zeros((spad,), f32)
    rows_sp = jnp.concatenate([rows[:N_SOCIAL], zs_i]).reshape(S_CHUNKS, 8, 128)
    cols_sp = jnp.concatenate([cols[:N_SOCIAL], zs_i]).reshape(S_CHUNKS, 8, 128)
    sw_p = jnp.concatenate([adj_values[:N_SOCIAL], zs_f])
    g_p = jnp.concatenate([g, zs_f])
    w2b = jnp.concatenate(
        [W2[:, 0], jnp.repeat(b2, 16), jnp.zeros((1280 - D - 16,), f32)])
    masked_sp = _edge(rows_sp, cols_sp, sw_p, g_p, w2b, P, Q)
    masked_s = masked_sp[:N_SOCIAL]

    pad = EP - N_EDGES
    ze_i = jnp.zeros((pad,), i32)
    ze_f = jnp.zeros((pad,), f32)
    rows_p = jnp.concatenate([rows, ze_i]).reshape(ROWS_E, 128)
    cols_p = jnp.concatenate([cols, ze_i])
    colsb = jnp.stack([cols_p, cols_p + N_NODES]).reshape(2, ROWS_E, 128)
    vals_p = jnp.concatenate([adj_values, ze_f])
    mvals_p = jnp.concatenate([masked_s, adj_values[N_SOCIAL:], ze_f])

    ego_cat = jnp.concatenate([ego[:, :H], ego[:, H:]], axis=0)
    x1 = _spmm(rows_p, colsb, vals_p, ego_cat)
    x2 = _spmm(rows_p, colsb, vals_p, x1)
    x3 = _spmm(rows_p, colsb, vals_p, x2)
    y1 = _spmm(rows_p, colsb, mvals_p, ego_cat)
    y2 = _spmm(rows_p, colsb, mvals_p, y1)
    y3 = _spmm(rows_p, colsb, mvals_p, y2)

    uo, um, po, pm, nm = _mgather(ego_cat, x1, x2, x3, y1, y2, y3,
                                  users, pos_items, neg_items)

    def fix(a):
        return a.reshape(2, B, H).transpose(1, 0, 2).reshape(B, D)

    loss, auc = _tail(users.reshape(B, 1), users.reshape(1, B),
                      pos_items.reshape(B, 1), pos_items.reshape(1, B),
                      fix(uo), fix(um), fix(po), fix(pm), fix(nm))
    return loss[0, 0], auc[0, 0]
